# initial kernel scaffold (unmeasured)
import jax
import jax.numpy as jnp
from jax import lax
from jax.experimental import pallas as pl
from jax.experimental.pallas import tpu as pltpu

N_DEV = 4
SQ = 2048
SKV_LOC = 2048
HQ = 8
DH = 128
DM = 1024
BLK = 64
SCALE = 0.08838834764831843
CHUNK = 512


def _qproj_body(x_ref, wq_ref, q_ref):
    xb = x_ref[0].astype(jnp.bfloat16)
    wb = wq_ref[:].astype(jnp.bfloat16)
    q = lax.dot_general(xb, wb, (((1,), (0,)), ((), ())),
                        preferred_element_type=jnp.float32)
    q_ref[:] = q.astype(jnp.bfloat16)


def _attn_body(q_ref, k_ref, v_ref, o_ref, l_ref, mask_ref):
    h = pl.program_id(0)
    p = lax.axis_index("i")

    @pl.when(h == 0)
    def _():
        for c in range(SQ // CHUNK):
            qi = c * CHUNK + lax.broadcasted_iota(
                jnp.int32, (CHUNK, SKV_LOC), 0)
            kj = lax.broadcasted_iota(jnp.int32, (CHUNK, SKV_LOC), 1)
            qb = qi // BLK
            kb = kj // BLK + p * (SKV_LOC // BLK)
            m = (qb == kb) | (kb == 0) | ((qb + kb) % 3 == 0)
            mask_ref[pl.ds(c * CHUNK, CHUNK), :] = m.astype(jnp.bfloat16)

    kh = k_ref[0, :, 0, :].astype(jnp.bfloat16)
    vh = v_ref[0, :, 0, :].astype(jnp.bfloat16)
    for c in range(SQ // CHUNK):
        qc = q_ref[pl.ds(c * CHUNK, CHUNK), :]
        s = lax.dot_general(qc, kh, (((1,), (1,)), ((), ())),
                            preferred_element_type=jnp.float32)
        w = jnp.exp(s * SCALE) * mask_ref[
            pl.ds(c * CHUNK, CHUNK), :].astype(jnp.float32)
        l_ref[pl.ds(c * CHUNK, CHUNK), :] = jnp.sum(w, axis=1, keepdims=True)
        o = lax.dot_general(w.astype(jnp.bfloat16), vh,
                            (((1,), (0,)), ((), ())),
                            preferred_element_type=jnp.float32)
        o_ref[pl.ds(c * CHUNK, CHUNK), :] = o.astype(jnp.bfloat16)


def _ring_body(o_ref, l_ref, wo_ref, out_ref,
               comm_o, comm_l, acc_o, acc_l, ctx_ref,
               send_o, recv_o, send_l, recv_l):
    p = lax.axis_index("i")
    left = lax.rem(p + N_DEV - 1, N_DEV)
    right = lax.rem(p + 1, N_DEV)

    comm_o[0] = o_ref[:]
    comm_l[0] = l_ref[:]
    acc_o[:] = o_ref[:].astype(jnp.float32)
    acc_l[:] = l_ref[:]

    barrier = pltpu.get_barrier_semaphore()
    pl.semaphore_signal(barrier, inc=1, device_id=(left,),
                        device_id_type=pl.DeviceIdType.MESH)
    pl.semaphore_signal(barrier, inc=1, device_id=(right,),
                        device_id_type=pl.DeviceIdType.MESH)
    pl.semaphore_wait(barrier, 2)

    for hop in range(N_DEV - 1):
        s_slot = hop % 2
        r_slot = (hop + 1) % 2
        rdma_o = pltpu.make_async_remote_copy(
            src_ref=comm_o.at[s_slot], dst_ref=comm_o.at[r_slot],
            send_sem=send_o.at[s_slot], recv_sem=recv_o.at[r_slot],
            device_id=(right,), device_id_type=pl.DeviceIdType.MESH)
        rdma_l = pltpu.make_async_remote_copy(
            src_ref=comm_l.at[s_slot], dst_ref=comm_l.at[r_slot],
            send_sem=send_l.at[s_slot], recv_sem=recv_l.at[r_slot],
            device_id=(right,), device_id_type=pl.DeviceIdType.MESH)
        rdma_o.start()
        rdma_l.start()
        rdma_o.wait()
        rdma_l.wait()
        acc_o[:] += comm_o[r_slot].astype(jnp.float32)
        acc_l[:] += comm_l[r_slot]

    inv = 1.0 / acc_l[:]
    for h in range(HQ):
        ctx_ref[:, h * DH:(h + 1) * DH] = (
            acc_o[:, h * DH:(h + 1) * DH] * inv[:, h:h + 1]
        ).astype(jnp.bfloat16)
    wo = wo_ref[:].astype(jnp.bfloat16)
    out_ref[0] = lax.dot_general(ctx_ref[:], wo, (((1,), (0,)), ((), ())),
                                 preferred_element_type=jnp.float32)


def kernel(x, Wq, K_ext, V_ext, Wo):
    Q = pl.pallas_call(
        _qproj_body,
        out_shape=jax.ShapeDtypeStruct((SQ, DM), jnp.bfloat16),
        in_specs=[pl.BlockSpec(memory_space=pltpu.VMEM),
                  pl.BlockSpec(memory_space=pltpu.VMEM)],
        out_specs=pl.BlockSpec(memory_space=pltpu.VMEM),
    )(x, Wq)

    o_p, l_p = pl.pallas_call(
        _attn_body,
        grid=(HQ,),
        in_specs=[
            pl.BlockSpec((SQ, DH), lambda h: (0, h)),
            pl.BlockSpec((1, SKV_LOC, 1, DH), lambda h: (0, 0, h, 0)),
            pl.BlockSpec((1, SKV_LOC, 1, DH), lambda h: (0, 0, h, 0)),
        ],
        out_specs=[
            pl.BlockSpec((SQ, DH), lambda h: (0, h)),
            pl.BlockSpec((SQ, 1), lambda h: (0, h)),
        ],
        out_shape=[
            jax.ShapeDtypeStruct((SQ, DM), jnp.bfloat16),
            jax.ShapeDtypeStruct((SQ, HQ), jnp.float32),
        ],
        scratch_shapes=[pltpu.VMEM((SQ, SKV_LOC), jnp.bfloat16)],
    )(Q, K_ext, V_ext)

    out = pl.pallas_call(
        _ring_body,
        out_shape=jax.ShapeDtypeStruct((1, SQ, DM), jnp.float32),
        in_specs=[pl.BlockSpec(memory_space=pltpu.VMEM),
                  pl.BlockSpec(memory_space=pltpu.VMEM),
                  pl.BlockSpec(memory_space=pltpu.VMEM)],
        out_specs=pl.BlockSpec(memory_space=pltpu.VMEM),
        scratch_shapes=[
            pltpu.VMEM((2, SQ, DM), jnp.bfloat16),
            pltpu.VMEM((2, SQ, HQ), jnp.float32),
            pltpu.VMEM((SQ, DM), jnp.float32),
            pltpu.VMEM((SQ, HQ), jnp.float32),
            pltpu.VMEM((SQ, DM), jnp.bfloat16),
            pltpu.SemaphoreType.DMA((2,)),
            pltpu.SemaphoreType.DMA((2,)),
            pltpu.SemaphoreType.DMA((2,)),
            pltpu.SemaphoreType.DMA((2,)),
        ],
        compiler_params=pltpu.CompilerParams(collective_id=0),
    )(o_p, l_p, Wo)
    return out


# baseline (device time: 267376 ns/iter reference)
import jax
import jax.numpy as jnp
from jax import lax
from jax.experimental import pallas as pl
from jax.experimental.pallas import tpu as pltpu

N_DEV = 4
SQ = 2048
SKV_LOC = 2048
HQ = 8
DH = 128
DM = 1024
BLK = 64
SCALE = 0.08838834764831843
CHUNK = 512


def _qproj_body(x_ref, wq_ref, q_ref):
    xb = x_ref[0].astype(jnp.bfloat16)
    wb = wq_ref[:].astype(jnp.bfloat16)
    q = lax.dot_general(xb, wb, (((1,), (0,)), ((), ())),
                        preferred_element_type=jnp.float32)
    q_ref[:] = q.astype(jnp.bfloat16)


def _attn_body(q_ref, k_ref, v_ref, o_ref, l_ref, mask_ref):
    p = lax.axis_index("i")

    for c in range(SQ // CHUNK):
        qi = c * CHUNK + lax.broadcasted_iota(
            jnp.int32, (CHUNK, SKV_LOC), 0)
        kj = lax.broadcasted_iota(jnp.int32, (CHUNK, SKV_LOC), 1)
        qb = qi // BLK
        kb = kj // BLK + p * (SKV_LOC // BLK)
        m = (qb == kb) | (kb == 0) | ((qb + kb) % 3 == 0)
        mask_ref[pl.ds(c * CHUNK, CHUNK), :] = m.astype(jnp.bfloat16)

    for h in range(HQ):
        kh = k_ref[0, :, h, :].astype(jnp.bfloat16)
        vh = v_ref[0, :, h, :].astype(jnp.bfloat16)
        for c in range(SQ // CHUNK):
            qc = q_ref[pl.ds(c * CHUNK, CHUNK), h * DH:(h + 1) * DH]
            s = lax.dot_general(qc, kh, (((1,), (1,)), ((), ())),
                                preferred_element_type=jnp.float32)
            w = jnp.exp(s * SCALE) * mask_ref[
                pl.ds(c * CHUNK, CHUNK), :].astype(jnp.float32)
            l_ref[pl.ds(c * CHUNK, CHUNK), h:h + 1] = jnp.sum(
                w, axis=1, keepdims=True)
            o = lax.dot_general(w.astype(jnp.bfloat16), vh,
                                (((1,), (0,)), ((), ())),
                                preferred_element_type=jnp.float32)
            o_ref[pl.ds(c * CHUNK, CHUNK),
                  h * DH:(h + 1) * DH] = o.astype(jnp.bfloat16)


def _ring_body(o_ref, l_ref, wo_ref, out_ref,
               comm_o, comm_l, acc_o, acc_l, ctx_ref,
               send_o, recv_o, send_l, recv_l):
    p = lax.axis_index("i")
    left = lax.rem(p + N_DEV - 1, N_DEV)
    right = lax.rem(p + 1, N_DEV)

    comm_o[0] = o_ref[:]
    comm_l[0] = l_ref[:]
    acc_o[:] = o_ref[:].astype(jnp.float32)
    acc_l[:] = l_ref[:]

    barrier = pltpu.get_barrier_semaphore()
    pl.semaphore_signal(barrier, inc=1, device_id=(left,),
                        device_id_type=pl.DeviceIdType.MESH)
    pl.semaphore_signal(barrier, inc=1, device_id=(right,),
                        device_id_type=pl.DeviceIdType.MESH)
    pl.semaphore_wait(barrier, 2)

    for hop in range(N_DEV - 1):
        s_slot = hop % 2
        r_slot = (hop + 1) % 2
        rdma_o = pltpu.make_async_remote_copy(
            src_ref=comm_o.at[s_slot], dst_ref=comm_o.at[r_slot],
            send_sem=send_o.at[s_slot], recv_sem=recv_o.at[r_slot],
            device_id=(right,), device_id_type=pl.DeviceIdType.MESH)
        rdma_l = pltpu.make_async_remote_copy(
            src_ref=comm_l.at[s_slot], dst_ref=comm_l.at[r_slot],
            send_sem=send_l.at[s_slot], recv_sem=recv_l.at[r_slot],
            device_id=(right,), device_id_type=pl.DeviceIdType.MESH)
        rdma_o.start()
        rdma_l.start()
        rdma_o.wait()
        rdma_l.wait()
        acc_o[:] += comm_o[r_slot].astype(jnp.float32)
        acc_l[:] += comm_l[r_slot]

    inv = 1.0 / acc_l[:]
    for h in range(HQ):
        ctx_ref[:, h * DH:(h + 1) * DH] = (
            acc_o[:, h * DH:(h + 1) * DH] * inv[:, h:h + 1]
        ).astype(jnp.bfloat16)
    wo = wo_ref[:].astype(jnp.bfloat16)
    out_ref[0] = lax.dot_general(ctx_ref[:], wo, (((1,), (0,)), ((), ())),
                                 preferred_element_type=jnp.float32)


def kernel(x, Wq, K_ext, V_ext, Wo):
    Q = pl.pallas_call(
        _qproj_body,
        out_shape=jax.ShapeDtypeStruct((SQ, DM), jnp.bfloat16),
        in_specs=[pl.BlockSpec(memory_space=pltpu.VMEM),
                  pl.BlockSpec(memory_space=pltpu.VMEM)],
        out_specs=pl.BlockSpec(memory_space=pltpu.VMEM),
    )(x, Wq)

    o_p, l_p = pl.pallas_call(
        _attn_body,
        in_specs=[pl.BlockSpec(memory_space=pltpu.VMEM),
                  pl.BlockSpec(memory_space=pltpu.VMEM),
                  pl.BlockSpec(memory_space=pltpu.VMEM)],
        out_specs=[pl.BlockSpec(memory_space=pltpu.VMEM),
                   pl.BlockSpec(memory_space=pltpu.VMEM)],
        out_shape=[
            jax.ShapeDtypeStruct((SQ, DM), jnp.bfloat16),
            jax.ShapeDtypeStruct((SQ, HQ), jnp.float32),
        ],
        scratch_shapes=[pltpu.VMEM((SQ, SKV_LOC), jnp.bfloat16)],
    )(Q, K_ext, V_ext)

    out = pl.pallas_call(
        _ring_body,
        out_shape=jax.ShapeDtypeStruct((1, SQ, DM), jnp.float32),
        in_specs=[pl.BlockSpec(memory_space=pltpu.VMEM),
                  pl.BlockSpec(memory_space=pltpu.VMEM),
                  pl.BlockSpec(memory_space=pltpu.VMEM)],
        out_specs=pl.BlockSpec(memory_space=pltpu.VMEM),
        scratch_shapes=[
            pltpu.VMEM((2, SQ, DM), jnp.bfloat16),
            pltpu.VMEM((2, SQ, HQ), jnp.float32),
            pltpu.VMEM((SQ, DM), jnp.float32),
            pltpu.VMEM((SQ, HQ), jnp.float32),
            pltpu.VMEM((SQ, DM), jnp.bfloat16),
            pltpu.SemaphoreType.DMA((2,)),
            pltpu.SemaphoreType.DMA((2,)),
            pltpu.SemaphoreType.DMA((2,)),
            pltpu.SemaphoreType.DMA((2,)),
        ],
        compiler_params=pltpu.CompilerParams(collective_id=0),
    )(o_p, l_p, Wo)
    return out


# device time: 172411 ns/iter; 1.5508x vs baseline; 1.5508x over previous
import jax
import jax.numpy as jnp
from jax import lax
from jax.experimental import pallas as pl
from jax.experimental.pallas import tpu as pltpu

N_DEV = 4
SQ = 2048
SKV_LOC = 2048
HQ = 8
DH = 128
DM = 1024
BLK = 64
SCALE = 0.08838834764831843
CHUNK = 512


def _qproj_body(x_ref, wq_ref, q_ref):
    xb = x_ref[0].astype(jnp.bfloat16)
    wb = wq_ref[:].astype(jnp.bfloat16)
    q = lax.dot_general(xb, wb, (((1,), (0,)), ((), ())),
                        preferred_element_type=jnp.float32)
    q_ref[:] = q.astype(jnp.bfloat16)


def _attn_body(q_ref, k_ref, v_ref, o_ref, l_ref, mask_ref):
    p = lax.axis_index("i")

    for c in range(SQ // CHUNK):
        qi = c * CHUNK + lax.broadcasted_iota(
            jnp.int32, (CHUNK, SKV_LOC), 0)
        kj = lax.broadcasted_iota(jnp.int32, (CHUNK, SKV_LOC), 1)
        qb = qi // BLK
        kb = kj // BLK + p * (SKV_LOC // BLK)
        m = (qb == kb) | (kb == 0) | ((qb + kb) % 3 == 0)
        mask_ref[pl.ds(c * CHUNK, CHUNK), :] = m.astype(jnp.bfloat16)

    for h in range(HQ):
        kh = k_ref[0, :, h, :].astype(jnp.bfloat16)
        vh = v_ref[0, :, h, :].astype(jnp.bfloat16)
        for c in range(SQ // CHUNK):
            qc = q_ref[pl.ds(c * CHUNK, CHUNK), h * DH:(h + 1) * DH]
            s = lax.dot_general(qc, kh, (((1,), (1,)), ((), ())),
                                preferred_element_type=jnp.float32)
            w = jnp.exp(s * SCALE) * mask_ref[
                pl.ds(c * CHUNK, CHUNK), :].astype(jnp.float32)
            l_ref[pl.ds(c * CHUNK, CHUNK), h:h + 1] = jnp.sum(
                w, axis=1, keepdims=True)
            o = lax.dot_general(w.astype(jnp.bfloat16), vh,
                                (((1,), (0,)), ((), ())),
                                preferred_element_type=jnp.float32)
            o_ref[pl.ds(c * CHUNK, CHUNK),
                  h * DH:(h + 1) * DH] = o.astype(jnp.bfloat16)


RCH = SQ // N_DEV


def _ring_body(o_ref, l_ref, wo_ref, out_ref,
               comm_o, comm_l, comm_g, ctx_ref,
               send_o, recv_o, send_l, recv_l, send_g, recv_g):
    p = lax.axis_index("i")
    left = lax.rem(p + N_DEV - 1, N_DEV)
    right = lax.rem(p + 1, N_DEV)

    comm_o[0] = o_ref[pl.ds(p * RCH, RCH), :]
    comm_l[0] = l_ref[pl.ds(p * RCH, RCH), :]

    barrier = pltpu.get_barrier_semaphore()
    pl.semaphore_signal(barrier, inc=1, device_id=(left,),
                        device_id_type=pl.DeviceIdType.MESH)
    pl.semaphore_signal(barrier, inc=1, device_id=(right,),
                        device_id_type=pl.DeviceIdType.MESH)
    pl.semaphore_wait(barrier, 2)

    fo = fl = None
    for t in range(N_DEV - 1):
        s_slot = t % 2
        r_slot = (t + 1) % 2
        rdma_o = pltpu.make_async_remote_copy(
            src_ref=comm_o.at[s_slot], dst_ref=comm_o.at[r_slot],
            send_sem=send_o.at[s_slot], recv_sem=recv_o.at[r_slot],
            device_id=(right,), device_id_type=pl.DeviceIdType.MESH)
        rdma_l = pltpu.make_async_remote_copy(
            src_ref=comm_l.at[s_slot], dst_ref=comm_l.at[r_slot],
            send_sem=send_l.at[s_slot], recv_sem=recv_l.at[r_slot],
            device_id=(right,), device_id_type=pl.DeviceIdType.MESH)
        rdma_o.start()
        rdma_l.start()
        rdma_o.wait()
        rdma_l.wait()
        c = lax.rem(p - t - 1 + N_DEV, N_DEV)
        my_o = o_ref[pl.ds(c * RCH, RCH), :].astype(jnp.float32)
        my_l = l_ref[pl.ds(c * RCH, RCH), :]
        if t < N_DEV - 2:
            comm_o[r_slot] = (
                comm_o[r_slot].astype(jnp.float32) + my_o
            ).astype(jnp.bfloat16)
            comm_l[r_slot] = comm_l[r_slot] + my_l
        else:
            fo = comm_o[r_slot].astype(jnp.float32) + my_o
            fl = comm_l[r_slot] + my_l

    inv = 1.0 / fl
    for h in range(HQ):
        ctx_ref[:, h * DH:(h + 1) * DH] = (
            fo[:, h * DH:(h + 1) * DH] * inv[:, h:h + 1]
        ).astype(jnp.bfloat16)
    wo = wo_ref[:].astype(jnp.bfloat16)
    my_out = lax.dot_general(ctx_ref[:], wo, (((1,), (0,)), ((), ())),
                             preferred_element_type=jnp.float32)
    my_chunk = lax.rem(p + 1, N_DEV)
    out_ref[0, pl.ds(my_chunk * RCH, RCH), :] = my_out
    comm_g[0] = my_out.astype(jnp.bfloat16)

    for g in range(N_DEV - 1):
        s_slot = g % 2
        r_slot = (g + 1) % 2
        rdma_g = pltpu.make_async_remote_copy(
            src_ref=comm_g.at[s_slot], dst_ref=comm_g.at[r_slot],
            send_sem=send_g.at[s_slot], recv_sem=recv_g.at[r_slot],
            device_id=(right,), device_id_type=pl.DeviceIdType.MESH)
        rdma_g.start()
        rdma_g.wait()
        oc = lax.rem(p - g + N_DEV, N_DEV)
        out_ref[0, pl.ds(oc * RCH, RCH), :] = comm_g[r_slot].astype(
            jnp.float32)


def kernel(x, Wq, K_ext, V_ext, Wo):
    Q = pl.pallas_call(
        _qproj_body,
        out_shape=jax.ShapeDtypeStruct((SQ, DM), jnp.bfloat16),
        in_specs=[pl.BlockSpec(memory_space=pltpu.VMEM),
                  pl.BlockSpec(memory_space=pltpu.VMEM)],
        out_specs=pl.BlockSpec(memory_space=pltpu.VMEM),
    )(x, Wq)

    o_p, l_p = pl.pallas_call(
        _attn_body,
        in_specs=[pl.BlockSpec(memory_space=pltpu.VMEM),
                  pl.BlockSpec(memory_space=pltpu.VMEM),
                  pl.BlockSpec(memory_space=pltpu.VMEM)],
        out_specs=[pl.BlockSpec(memory_space=pltpu.VMEM),
                   pl.BlockSpec(memory_space=pltpu.VMEM)],
        out_shape=[
            jax.ShapeDtypeStruct((SQ, DM), jnp.bfloat16),
            jax.ShapeDtypeStruct((SQ, HQ), jnp.float32),
        ],
        scratch_shapes=[pltpu.VMEM((SQ, SKV_LOC), jnp.bfloat16)],
    )(Q, K_ext, V_ext)

    out = pl.pallas_call(
        _ring_body,
        out_shape=jax.ShapeDtypeStruct((1, SQ, DM), jnp.float32),
        in_specs=[pl.BlockSpec(memory_space=pltpu.VMEM),
                  pl.BlockSpec(memory_space=pltpu.VMEM),
                  pl.BlockSpec(memory_space=pltpu.VMEM)],
        out_specs=pl.BlockSpec(memory_space=pltpu.VMEM),
        scratch_shapes=[
            pltpu.VMEM((2, RCH, DM), jnp.bfloat16),
            pltpu.VMEM((2, RCH, HQ), jnp.float32),
            pltpu.VMEM((2, RCH, DM), jnp.bfloat16),
            pltpu.VMEM((RCH, DM), jnp.bfloat16),
            pltpu.SemaphoreType.DMA((2,)),
            pltpu.SemaphoreType.DMA((2,)),
            pltpu.SemaphoreType.DMA((2,)),
            pltpu.SemaphoreType.DMA((2,)),
            pltpu.SemaphoreType.DMA((2,)),
            pltpu.SemaphoreType.DMA((2,)),
        ],
        compiler_params=pltpu.CompilerParams(collective_id=0),
    )(o_p, l_p, Wo)
    return out


# device time: 148394 ns/iter; 1.8018x vs baseline; 1.1618x over previous
import jax
import jax.numpy as jnp
from jax import lax
from jax.experimental import pallas as pl
from jax.experimental.pallas import tpu as pltpu

N_DEV = 4
SQ = 2048
SKV_LOC = 2048
HQ = 8
DH = 128
DM = 1024
BLK = 64
SCALE = 0.08838834764831843
RCH = SQ // N_DEV


def _qproj_body(x_ref, wq_ref, q3_ref):
    xb = x_ref[0].astype(jnp.bfloat16)
    wb = wq_ref[:].astype(jnp.bfloat16)
    q = lax.dot_general(xb, wb, (((1,), (0,)), ((), ())),
                        preferred_element_type=jnp.float32)
    qb16 = q.astype(jnp.bfloat16)
    for h in range(HQ):
        q3_ref[h] = qb16[:, h * DH:(h + 1) * DH]


def _kvprep_body(k_ref, v_ref, kt_ref, vt_ref):
    for h in range(HQ):
        kt_ref[h] = k_ref[0, :, h, :].astype(jnp.bfloat16)
        vt_ref[h] = v_ref[0, :, h, :].astype(jnp.bfloat16)


def _fused_body(q3_ref, kt_ref, vt_ref, wo_ref, out_ref,
                mask_ref, comm_o, comm_l, comm_g, work_o, work_l, ctx_ref,
                send_o, recv_o, send_l, recv_l, send_g, recv_g):
    p = lax.axis_index("i")
    left = lax.rem(p + N_DEV - 1, N_DEV)
    right = lax.rem(p + 1, N_DEV)

    barrier = pltpu.get_barrier_semaphore()
    pl.semaphore_signal(barrier, inc=1, device_id=(left,),
                        device_id_type=pl.DeviceIdType.MESH)
    pl.semaphore_signal(barrier, inc=1, device_id=(right,),
                        device_id_type=pl.DeviceIdType.MESH)

    def compute_chunk(c):
        row0 = c * RCH
        qi = row0 + lax.broadcasted_iota(jnp.int32, (RCH, SKV_LOC), 0)
        kj = lax.broadcasted_iota(jnp.int32, (RCH, SKV_LOC), 1)
        qb = qi // BLK
        kb = kj // BLK + p * (SKV_LOC // BLK)
        m = (qb == kb) | (kb == 0) | ((qb + kb) % 3 == 0)
        mask_ref[:] = m.astype(jnp.bfloat16)
        for h in range(HQ):
            qc = q3_ref[h, pl.ds(row0, RCH), :]
            s = lax.dot_general(qc, kt_ref[h], (((1,), (1,)), ((), ())),
                                preferred_element_type=jnp.float32)
            w = jnp.exp(s * SCALE) * mask_ref[:].astype(jnp.float32)
            work_l[:, h:h + 1] = jnp.sum(w, axis=1, keepdims=True)
            o = lax.dot_general(w.astype(jnp.bfloat16), vt_ref[h],
                                (((1,), (0,)), ((), ())),
                                preferred_element_type=jnp.float32)
            work_o[:, h * DH:(h + 1) * DH] = o

    compute_chunk(p)
    comm_o[0] = work_o[:].astype(jnp.bfloat16)
    comm_l[0] = work_l[:]
    pl.semaphore_wait(barrier, 2)

    fo = fl = None
    for t in range(N_DEV - 1):
        rdma_o = pltpu.make_async_remote_copy(
            src_ref=comm_o.at[t], dst_ref=comm_o.at[t + 1],
            send_sem=send_o.at[t], recv_sem=recv_o.at[t + 1],
            device_id=(right,), device_id_type=pl.DeviceIdType.MESH)
        rdma_l = pltpu.make_async_remote_copy(
            src_ref=comm_l.at[t], dst_ref=comm_l.at[t + 1],
            send_sem=send_l.at[t], recv_sem=recv_l.at[t + 1],
            device_id=(right,), device_id_type=pl.DeviceIdType.MESH)
        rdma_o.start()
        rdma_l.start()
        c = lax.rem(p - t - 1 + N_DEV, N_DEV)
        compute_chunk(c)
        rdma_o.wait()
        rdma_l.wait()
        if t < N_DEV - 2:
            comm_o[t + 1] = (
                comm_o[t + 1].astype(jnp.float32) + work_o[:]
            ).astype(jnp.bfloat16)
            comm_l[t + 1] = comm_l[t + 1] + work_l[:]
        else:
            fo = comm_o[t + 1].astype(jnp.float32) + work_o[:]
            fl = comm_l[t + 1] + work_l[:]

    inv = 1.0 / fl
    for h in range(HQ):
        ctx_ref[:, h * DH:(h + 1) * DH] = (
            fo[:, h * DH:(h + 1) * DH] * inv[:, h:h + 1]
        ).astype(jnp.bfloat16)
    wo = wo_ref[:].astype(jnp.bfloat16)
    my_out = lax.dot_general(ctx_ref[:], wo, (((1,), (0,)), ((), ())),
                             preferred_element_type=jnp.float32)
    my_chunk = lax.rem(p + 1, N_DEV)
    out_ref[0, pl.ds(my_chunk * RCH, RCH), :] = my_out
    comm_g[0] = my_out.astype(jnp.bfloat16)

    for g in range(N_DEV - 1):
        rdma_g = pltpu.make_async_remote_copy(
            src_ref=comm_g.at[g], dst_ref=comm_g.at[g + 1],
            send_sem=send_g.at[g], recv_sem=recv_g.at[g + 1],
            device_id=(right,), device_id_type=pl.DeviceIdType.MESH)
        rdma_g.start()
        rdma_g.wait()
        oc = lax.rem(p - g + N_DEV, N_DEV)
        out_ref[0, pl.ds(oc * RCH, RCH), :] = comm_g[g + 1].astype(
            jnp.float32)


def kernel(x, Wq, K_ext, V_ext, Wo):
    Q3 = pl.pallas_call(
        _qproj_body,
        out_shape=jax.ShapeDtypeStruct((HQ, SQ, DH), jnp.bfloat16),
        in_specs=[pl.BlockSpec(memory_space=pltpu.VMEM),
                  pl.BlockSpec(memory_space=pltpu.VMEM)],
        out_specs=pl.BlockSpec(memory_space=pltpu.VMEM),
    )(x, Wq)

    K_t, V_t = pl.pallas_call(
        _kvprep_body,
        out_shape=[
            jax.ShapeDtypeStruct((HQ, SKV_LOC, DH), jnp.bfloat16),
            jax.ShapeDtypeStruct((HQ, SKV_LOC, DH), jnp.bfloat16),
        ],
        in_specs=[pl.BlockSpec(memory_space=pltpu.VMEM),
                  pl.BlockSpec(memory_space=pltpu.VMEM)],
        out_specs=[pl.BlockSpec(memory_space=pltpu.VMEM),
                   pl.BlockSpec(memory_space=pltpu.VMEM)],
    )(K_ext, V_ext)

    out = pl.pallas_call(
        _fused_body,
        out_shape=jax.ShapeDtypeStruct((1, SQ, DM), jnp.float32),
        in_specs=[pl.BlockSpec(memory_space=pltpu.VMEM),
                  pl.BlockSpec(memory_space=pltpu.VMEM),
                  pl.BlockSpec(memory_space=pltpu.VMEM),
                  pl.BlockSpec(memory_space=pltpu.VMEM)],
        out_specs=pl.BlockSpec(memory_space=pltpu.VMEM),
        scratch_shapes=[
            pltpu.VMEM((RCH, SKV_LOC), jnp.bfloat16),
            pltpu.VMEM((N_DEV, RCH, DM), jnp.bfloat16),
            pltpu.VMEM((N_DEV, RCH, HQ), jnp.float32),
            pltpu.VMEM((N_DEV, RCH, DM), jnp.bfloat16),
            pltpu.VMEM((RCH, DM), jnp.float32),
            pltpu.VMEM((RCH, HQ), jnp.float32),
            pltpu.VMEM((RCH, DM), jnp.bfloat16),
            pltpu.SemaphoreType.DMA((N_DEV,)),
            pltpu.SemaphoreType.DMA((N_DEV,)),
            pltpu.SemaphoreType.DMA((N_DEV,)),
            pltpu.SemaphoreType.DMA((N_DEV,)),
            pltpu.SemaphoreType.DMA((N_DEV,)),
            pltpu.SemaphoreType.DMA((N_DEV,)),
        ],
        compiler_params=pltpu.CompilerParams(collective_id=0),
    )(Q3, K_t, V_t, Wo)
    return out


# device time: 134790 ns/iter; 1.9836x vs baseline; 1.1009x over previous
import jax
import jax.numpy as jnp
from jax import lax
from jax.experimental import pallas as pl
from jax.experimental.pallas import tpu as pltpu

N_DEV = 4
SQ = 2048
SKV_LOC = 2048
HQ = 8
DH = 128
DM = 1024
BLK = 64
SCALE = 0.08838834764831843
RCH = SQ // N_DEV


def _qproj_body(x_ref, wq_ref, q3_ref):
    xb = x_ref[0].astype(jnp.bfloat16)
    wb = wq_ref[:].astype(jnp.bfloat16)
    q = lax.dot_general(xb, wb, (((1,), (0,)), ((), ())),
                        preferred_element_type=jnp.float32)
    qb16 = q.astype(jnp.bfloat16)
    for h in range(HQ):
        q3_ref[h] = qb16[:, h * DH:(h + 1) * DH]


def _kvprep_body(k_ref, v_ref, kt_ref, vt_ref):
    for h in range(HQ):
        kt_ref[h] = k_ref[0, :, h, :].astype(jnp.bfloat16)
        vt_ref[h] = v_ref[0, :, h, :].astype(jnp.bfloat16)


def _fused_body(q3_ref, kt_ref, vt_ref, wo_ref, out_ref,
                mask_ref, comm_o, comm_l, comm_g, work_o, work_l, ctx_ref,
                send_o, recv_o, send_l, recv_l, send_g, recv_g):
    p = lax.axis_index("i")
    left = lax.rem(p + N_DEV - 1, N_DEV)
    right = lax.rem(p + 1, N_DEV)

    barrier = pltpu.get_barrier_semaphore()
    pl.semaphore_signal(barrier, inc=1, device_id=(left,),
                        device_id_type=pl.DeviceIdType.MESH)
    pl.semaphore_signal(barrier, inc=1, device_id=(right,),
                        device_id_type=pl.DeviceIdType.MESH)

    def compute_chunk(c):
        row0 = c * RCH
        qi = row0 + lax.broadcasted_iota(jnp.int32, (RCH, SKV_LOC), 0)
        kj = lax.broadcasted_iota(jnp.int32, (RCH, SKV_LOC), 1)
        qb = qi // BLK
        kb = kj // BLK + p * (SKV_LOC // BLK)
        m = (qb == kb) | (kb == 0) | ((qb + kb) % 3 == 0)
        mask_ref[:] = m.astype(jnp.bfloat16)
        for h in range(HQ):
            qc = q3_ref[h, pl.ds(row0, RCH), :]
            s = lax.dot_general(qc, kt_ref[h], (((1,), (1,)), ((), ())),
                                preferred_element_type=jnp.float32)
            w = jnp.exp(s * SCALE) * mask_ref[:].astype(jnp.float32)
            work_l[:, h:h + 1] = jnp.sum(w, axis=1, keepdims=True)
            o = lax.dot_general(w.astype(jnp.bfloat16), vt_ref[h],
                                (((1,), (0,)), ((), ())),
                                preferred_element_type=jnp.float32)
            work_o[:, h * DH:(h + 1) * DH] = o

    compute_chunk(p)
    comm_o[0] = work_o[:].astype(jnp.bfloat16)
    comm_l[0] = work_l[:]
    pl.semaphore_wait(barrier, 2)

    fo = fl = None
    for t in range(N_DEV - 1):
        rdma_o = pltpu.make_async_remote_copy(
            src_ref=comm_o.at[t], dst_ref=comm_o.at[t + 1],
            send_sem=send_o.at[t], recv_sem=recv_o.at[t + 1],
            device_id=(right,), device_id_type=pl.DeviceIdType.MESH)
        rdma_l = pltpu.make_async_remote_copy(
            src_ref=comm_l.at[t], dst_ref=comm_l.at[t + 1],
            send_sem=send_l.at[t], recv_sem=recv_l.at[t + 1],
            device_id=(right,), device_id_type=pl.DeviceIdType.MESH)
        rdma_o.start()
        rdma_l.start()
        c = lax.rem(p - t - 1 + N_DEV, N_DEV)
        compute_chunk(c)
        rdma_o.wait()
        rdma_l.wait()
        if t < N_DEV - 2:
            comm_o[t + 1] = (
                comm_o[t + 1].astype(jnp.float32) + work_o[:]
            ).astype(jnp.bfloat16)
            comm_l[t + 1] = comm_l[t + 1] + work_l[:]
        else:
            fo = comm_o[t + 1].astype(jnp.float32) + work_o[:]
            fl = comm_l[t + 1] + work_l[:]

    inv = 1.0 / fl
    for h in range(HQ):
        ctx_ref[:, h * DH:(h + 1) * DH] = (
            fo[:, h * DH:(h + 1) * DH] * inv[:, h:h + 1]
        ).astype(jnp.bfloat16)
    wo = wo_ref[:].astype(jnp.bfloat16)
    my_out = lax.dot_general(ctx_ref[:], wo, (((1,), (0,)), ((), ())),
                             preferred_element_type=jnp.float32)
    my_chunk = lax.rem(p + 1, N_DEV)
    out_ref[0, pl.ds(my_chunk * RCH, RCH), :] = my_out
    comm_g[0] = my_out.astype(jnp.bfloat16)

    ag_r = pltpu.make_async_remote_copy(
        src_ref=comm_g.at[0], dst_ref=comm_g.at[1],
        send_sem=send_g.at[0], recv_sem=recv_g.at[1],
        device_id=(right,), device_id_type=pl.DeviceIdType.MESH)
    ag_l = pltpu.make_async_remote_copy(
        src_ref=comm_g.at[0], dst_ref=comm_g.at[2],
        send_sem=send_g.at[1], recv_sem=recv_g.at[2],
        device_id=(left,), device_id_type=pl.DeviceIdType.MESH)
    ag_r.start()
    ag_l.start()
    ag_l.wait()
    ag_f = pltpu.make_async_remote_copy(
        src_ref=comm_g.at[2], dst_ref=comm_g.at[3],
        send_sem=send_g.at[2], recv_sem=recv_g.at[3],
        device_id=(left,), device_id_type=pl.DeviceIdType.MESH)
    ag_f.start()
    out_ref[0, pl.ds(lax.rem(p + 2, N_DEV) * RCH, RCH), :] = (
        comm_g[2].astype(jnp.float32))
    ag_r.wait()
    out_ref[0, pl.ds(p * RCH, RCH), :] = comm_g[1].astype(jnp.float32)
    ag_f.wait()
    out_ref[0, pl.ds(lax.rem(p + 3, N_DEV) * RCH, RCH), :] = (
        comm_g[3].astype(jnp.float32))


def kernel(x, Wq, K_ext, V_ext, Wo):
    Q3 = pl.pallas_call(
        _qproj_body,
        out_shape=jax.ShapeDtypeStruct((HQ, SQ, DH), jnp.bfloat16),
        in_specs=[pl.BlockSpec(memory_space=pltpu.VMEM),
                  pl.BlockSpec(memory_space=pltpu.VMEM)],
        out_specs=pl.BlockSpec(memory_space=pltpu.VMEM),
    )(x, Wq)

    K_t, V_t = pl.pallas_call(
        _kvprep_body,
        out_shape=[
            jax.ShapeDtypeStruct((HQ, SKV_LOC, DH), jnp.bfloat16),
            jax.ShapeDtypeStruct((HQ, SKV_LOC, DH), jnp.bfloat16),
        ],
        in_specs=[pl.BlockSpec(memory_space=pltpu.VMEM),
                  pl.BlockSpec(memory_space=pltpu.VMEM)],
        out_specs=[pl.BlockSpec(memory_space=pltpu.VMEM),
                   pl.BlockSpec(memory_space=pltpu.VMEM)],
    )(K_ext, V_ext)

    out = pl.pallas_call(
        _fused_body,
        out_shape=jax.ShapeDtypeStruct((1, SQ, DM), jnp.float32),
        in_specs=[pl.BlockSpec(memory_space=pltpu.VMEM),
                  pl.BlockSpec(memory_space=pltpu.VMEM),
                  pl.BlockSpec(memory_space=pltpu.VMEM),
                  pl.BlockSpec(memory_space=pltpu.VMEM)],
        out_specs=pl.BlockSpec(memory_space=pltpu.VMEM),
        scratch_shapes=[
            pltpu.VMEM((RCH, SKV_LOC), jnp.bfloat16),
            pltpu.VMEM((N_DEV, RCH, DM), jnp.bfloat16),
            pltpu.VMEM((N_DEV, RCH, HQ), jnp.float32),
            pltpu.VMEM((N_DEV, RCH, DM), jnp.bfloat16),
            pltpu.VMEM((RCH, DM), jnp.float32),
            pltpu.VMEM((RCH, HQ), jnp.float32),
            pltpu.VMEM((RCH, DM), jnp.bfloat16),
            pltpu.SemaphoreType.DMA((N_DEV,)),
            pltpu.SemaphoreType.DMA((N_DEV,)),
            pltpu.SemaphoreType.DMA((N_DEV,)),
            pltpu.SemaphoreType.DMA((N_DEV,)),
            pltpu.SemaphoreType.DMA((N_DEV,)),
            pltpu.SemaphoreType.DMA((N_DEV,)),
        ],
        compiler_params=pltpu.CompilerParams(collective_id=0),
    )(Q3, K_t, V_t, Wo)
    return out


# device time: 116336 ns/iter; 2.2983x vs baseline; 1.1586x over previous
import jax
import jax.numpy as jnp
from jax import lax
from jax.experimental import pallas as pl
from jax.experimental.pallas import tpu as pltpu

N_DEV = 4
SQ = 2048
SKV_LOC = 2048
HQ = 8
DH = 128
DM = 1024
BLK = 64
SCALE = 0.08838834764831843
RCH = SQ // N_DEV
HCH = RCH // 2


def _qproj_body(x_ref, wq_ref, q3_ref):
    xb = x_ref[0].astype(jnp.bfloat16)
    wb = wq_ref[:].astype(jnp.bfloat16)
    q = lax.dot_general(xb, wb, (((1,), (0,)), ((), ())),
                        preferred_element_type=jnp.float32)
    qb16 = q.astype(jnp.bfloat16)
    for h in range(HQ):
        q3_ref[h] = qb16[:, h * DH:(h + 1) * DH]


def _kvprep_body(k_ref, v_ref, kt_ref, vt_ref):
    kt_ref[:] = jnp.transpose(k_ref[0], (1, 0, 2)).astype(jnp.bfloat16)
    vt_ref[:] = jnp.transpose(v_ref[0], (1, 0, 2)).astype(jnp.bfloat16)


def _fused_body(q3_ref, kt_ref, vt_ref, wo_ref, out_ref,
                mask_ref, comm_o, comm_l, comm_g, work_o, work_l, ctx_ref,
                send_o, recv_o, send_l, recv_l, send_g, recv_g):
    p = lax.axis_index("i")
    left = lax.rem(p + N_DEV - 1, N_DEV)
    right = lax.rem(p + 1, N_DEV)

    barrier = pltpu.get_barrier_semaphore()
    pl.semaphore_signal(barrier, inc=1, device_id=(left,),
                        device_id_type=pl.DeviceIdType.MESH)
    pl.semaphore_signal(barrier, inc=1, device_id=(right,),
                        device_id_type=pl.DeviceIdType.MESH)

    def compute_chunk(c):
        row0 = c * RCH
        qi = row0 + lax.broadcasted_iota(jnp.int32, (RCH, SKV_LOC), 0)
        kj = lax.broadcasted_iota(jnp.int32, (RCH, SKV_LOC), 1)
        qb = qi // BLK
        kb = kj // BLK + p * (SKV_LOC // BLK)
        m = (qb == kb) | (kb == 0) | ((qb + kb) % 3 == 0)
        mask_ref[:] = m.astype(jnp.bfloat16)
        for h in range(HQ):
            qc = q3_ref[h, pl.ds(row0, RCH), :]
            s = lax.dot_general(qc, kt_ref[h], (((1,), (1,)), ((), ())),
                                preferred_element_type=jnp.float32)
            w = jnp.exp((s * SCALE).astype(jnp.bfloat16)) * mask_ref[:]
            work_l[:, h:h + 1] = jnp.sum(w, axis=1, keepdims=True,
                                         dtype=jnp.float32)
            o = lax.dot_general(w, vt_ref[h],
                                (((1,), (0,)), ((), ())),
                                preferred_element_type=jnp.float32)
            work_o[:, h * DH:(h + 1) * DH] = o

    compute_chunk(p)
    comm_o[0] = work_o[:].astype(jnp.bfloat16)
    comm_l[0] = work_l[:]
    pl.semaphore_wait(barrier, 2)

    fo = fl = None
    for t in range(N_DEV - 1):
        rdma_o = pltpu.make_async_remote_copy(
            src_ref=comm_o.at[t], dst_ref=comm_o.at[t + 1],
            send_sem=send_o.at[t], recv_sem=recv_o.at[t + 1],
            device_id=(right,), device_id_type=pl.DeviceIdType.MESH)
        rdma_l = pltpu.make_async_remote_copy(
            src_ref=comm_l.at[t], dst_ref=comm_l.at[t + 1],
            send_sem=send_l.at[t], recv_sem=recv_l.at[t + 1],
            device_id=(right,), device_id_type=pl.DeviceIdType.MESH)
        rdma_o.start()
        rdma_l.start()
        c = lax.rem(p - t - 1 + N_DEV, N_DEV)
        compute_chunk(c)
        rdma_o.wait()
        rdma_l.wait()
        if t < N_DEV - 2:
            comm_o[t + 1] = (
                comm_o[t + 1].astype(jnp.float32) + work_o[:]
            ).astype(jnp.bfloat16)
            comm_l[t + 1] = comm_l[t + 1] + work_l[:]
        else:
            fo = comm_o[t + 1].astype(jnp.float32) + work_o[:]
            fl = comm_l[t + 1] + work_l[:]

    inv = 1.0 / fl
    for h in range(HQ):
        ctx_ref[:, h * DH:(h + 1) * DH] = (
            fo[:, h * DH:(h + 1) * DH] * inv[:, h:h + 1]
        ).astype(jnp.bfloat16)
    wo = wo_ref[:].astype(jnp.bfloat16)
    my_out = lax.dot_general(ctx_ref[:], wo, (((1,), (0,)), ((), ())),
                             preferred_element_type=jnp.float32)
    my_chunk = lax.rem(p + 1, N_DEV)
    out_ref[0, pl.ds(my_chunk * RCH, RCH), :] = my_out
    comm_g[0] = my_out.astype(jnp.bfloat16)

    ag_r = pltpu.make_async_remote_copy(
        src_ref=comm_g.at[0], dst_ref=comm_g.at[1],
        send_sem=send_g.at[0], recv_sem=recv_g.at[1],
        device_id=(right,), device_id_type=pl.DeviceIdType.MESH)
    ag_l = pltpu.make_async_remote_copy(
        src_ref=comm_g.at[0], dst_ref=comm_g.at[2],
        send_sem=send_g.at[1], recv_sem=recv_g.at[2],
        device_id=(left,), device_id_type=pl.DeviceIdType.MESH)
    ag_r.start()
    ag_l.start()
    ag_l.wait()
    ag_fl = pltpu.make_async_remote_copy(
        src_ref=comm_g.at[2, pl.ds(0, HCH)],
        dst_ref=comm_g.at[3, pl.ds(0, HCH)],
        send_sem=send_g.at[2], recv_sem=recv_g.at[3],
        device_id=(left,), device_id_type=pl.DeviceIdType.MESH)
    ag_fl.start()
    out_ref[0, pl.ds(lax.rem(p + 2, N_DEV) * RCH, RCH), :] = (
        comm_g[2].astype(jnp.float32))
    ag_r.wait()
    ag_fr = pltpu.make_async_remote_copy(
        src_ref=comm_g.at[1, pl.ds(HCH, HCH)],
        dst_ref=comm_g.at[3, pl.ds(HCH, HCH)],
        send_sem=send_g.at[3], recv_sem=recv_g.at[4],
        device_id=(right,), device_id_type=pl.DeviceIdType.MESH)
    ag_fr.start()
    out_ref[0, pl.ds(p * RCH, RCH), :] = comm_g[1].astype(jnp.float32)
    ag_fl.wait()
    ag_fr.wait()
    out_ref[0, pl.ds(lax.rem(p + 3, N_DEV) * RCH, RCH), :] = (
        comm_g[3].astype(jnp.float32))


def kernel(x, Wq, K_ext, V_ext, Wo):
    Q3 = pl.pallas_call(
        _qproj_body,
        out_shape=jax.ShapeDtypeStruct((HQ, SQ, DH), jnp.bfloat16),
        in_specs=[pl.BlockSpec(memory_space=pltpu.VMEM),
                  pl.BlockSpec(memory_space=pltpu.VMEM)],
        out_specs=pl.BlockSpec(memory_space=pltpu.VMEM),
    )(x, Wq)

    K_t, V_t = pl.pallas_call(
        _kvprep_body,
        out_shape=[
            jax.ShapeDtypeStruct((HQ, SKV_LOC, DH), jnp.bfloat16),
            jax.ShapeDtypeStruct((HQ, SKV_LOC, DH), jnp.bfloat16),
        ],
        in_specs=[pl.BlockSpec(memory_space=pltpu.VMEM),
                  pl.BlockSpec(memory_space=pltpu.VMEM)],
        out_specs=[pl.BlockSpec(memory_space=pltpu.VMEM),
                   pl.BlockSpec(memory_space=pltpu.VMEM)],
    )(K_ext, V_ext)

    out = pl.pallas_call(
        _fused_body,
        out_shape=jax.ShapeDtypeStruct((1, SQ, DM), jnp.float32),
        in_specs=[pl.BlockSpec(memory_space=pltpu.VMEM),
                  pl.BlockSpec(memory_space=pltpu.VMEM),
                  pl.BlockSpec(memory_space=pltpu.VMEM),
                  pl.BlockSpec(memory_space=pltpu.VMEM)],
        out_specs=pl.BlockSpec(memory_space=pltpu.VMEM),
        scratch_shapes=[
            pltpu.VMEM((RCH, SKV_LOC), jnp.bfloat16),
            pltpu.VMEM((N_DEV, RCH, DM), jnp.bfloat16),
            pltpu.VMEM((N_DEV, RCH, HQ), jnp.float32),
            pltpu.VMEM((N_DEV, RCH, DM), jnp.bfloat16),
            pltpu.VMEM((RCH, DM), jnp.float32),
            pltpu.VMEM((RCH, HQ), jnp.float32),
            pltpu.VMEM((RCH, DM), jnp.bfloat16),
            pltpu.SemaphoreType.DMA((N_DEV,)),
            pltpu.SemaphoreType.DMA((N_DEV,)),
            pltpu.SemaphoreType.DMA((N_DEV,)),
            pltpu.SemaphoreType.DMA((N_DEV,)),
            pltpu.SemaphoreType.DMA((8,)),
            pltpu.SemaphoreType.DMA((8,)),
        ],
        compiler_params=pltpu.CompilerParams(collective_id=0),
    )(Q3, K_t, V_t, Wo)
    return out


# device time: 112156 ns/iter; 2.3840x vs baseline; 1.0373x over previous
import jax
import jax.numpy as jnp
from jax import lax
from jax.experimental import pallas as pl
from jax.experimental.pallas import tpu as pltpu

N_DEV = 4
SQ = 2048
SKV_LOC = 2048
HQ = 8
DH = 128
DM = 1024
BLK = 64
SCALE = 0.08838834764831843
RCH = SQ // N_DEV
HCH = RCH // 2


def _qproj_body(x_ref, wq_ref, q3_ref):
    xb = x_ref[0].astype(jnp.bfloat16)
    wb = wq_ref[:].astype(jnp.bfloat16)
    q = lax.dot_general(xb, wb, (((1,), (0,)), ((), ())),
                        preferred_element_type=jnp.float32)
    qb16 = q.astype(jnp.bfloat16)
    for h in range(HQ):
        q3_ref[h] = qb16[:, h * DH:(h + 1) * DH]


def _kvprep_body(k_ref, v_ref, kt_ref, vt_ref):
    kt_ref[:] = jnp.transpose(k_ref[0], (1, 0, 2)).astype(jnp.bfloat16)
    vt_ref[:] = jnp.transpose(v_ref[0], (1, 0, 2)).astype(jnp.bfloat16)


def _fused_body(q3_ref, kt_ref, vt_ref, wo_ref, out_ref,
                mask_ref, comm_o, comm_l, comm_g, work_o, work_l, ctx_ref,
                send_o, recv_o, send_l, recv_l, send_g, recv_g):
    p = lax.axis_index("i")
    left = lax.rem(p + N_DEV - 1, N_DEV)
    right = lax.rem(p + 1, N_DEV)

    barrier = pltpu.get_barrier_semaphore()
    pl.semaphore_signal(barrier, inc=1, device_id=(left,),
                        device_id_type=pl.DeviceIdType.MESH)
    pl.semaphore_signal(barrier, inc=1, device_id=(right,),
                        device_id_type=pl.DeviceIdType.MESH)

    def compute_half(c, j):
        off = j * HCH
        row0 = c * RCH + off
        qi = row0 + lax.broadcasted_iota(jnp.int32, (HCH, SKV_LOC), 0)
        kj = lax.broadcasted_iota(jnp.int32, (HCH, SKV_LOC), 1)
        qb = qi // BLK
        kb = kj // BLK + p * (SKV_LOC // BLK)
        m = (qb == kb) | (kb == 0) | ((qb + kb) % 3 == 0)
        mask_ref[:] = m.astype(jnp.bfloat16)
        for h in range(HQ):
            qc = q3_ref[h, pl.ds(row0, HCH), :]
            s = lax.dot_general(qc, kt_ref[h], (((1,), (1,)), ((), ())),
                                preferred_element_type=jnp.float32)
            w = jnp.exp((s * SCALE).astype(jnp.bfloat16)) * mask_ref[:]
            work_l[off:off + HCH, h:h + 1] = jnp.sum(
                w, axis=1, keepdims=True, dtype=jnp.float32)
            o = lax.dot_general(w, vt_ref[h],
                                (((1,), (0,)), ((), ())),
                                preferred_element_type=jnp.float32)
            work_o[off:off + HCH, h * DH:(h + 1) * DH] = o

    def rs_half(t, j):
        r0 = j * HCH
        ro = pltpu.make_async_remote_copy(
            src_ref=comm_o.at[t, pl.ds(r0, HCH)],
            dst_ref=comm_o.at[t + 1, pl.ds(r0, HCH)],
            send_sem=send_o.at[4 * j + t], recv_sem=recv_o.at[4 * j + t + 1],
            device_id=(right,), device_id_type=pl.DeviceIdType.MESH)
        rl = pltpu.make_async_remote_copy(
            src_ref=comm_l.at[t, pl.ds(r0, HCH)],
            dst_ref=comm_l.at[t + 1, pl.ds(r0, HCH)],
            send_sem=send_l.at[4 * j + t], recv_sem=recv_l.at[4 * j + t + 1],
            device_id=(right,), device_id_type=pl.DeviceIdType.MESH)
        ro.start()
        rl.start()
        return ro, rl

    compute_half(p, 0)
    comm_o[0, pl.ds(0, HCH)] = work_o[0:HCH, :].astype(jnp.bfloat16)
    comm_l[0, pl.ds(0, HCH)] = work_l[0:HCH, :]
    pl.semaphore_wait(barrier, 2)
    cur_a = rs_half(0, 0)
    compute_half(p, 1)
    comm_o[0, pl.ds(HCH, HCH)] = work_o[HCH:RCH, :].astype(jnp.bfloat16)
    comm_l[0, pl.ds(HCH, HCH)] = work_l[HCH:RCH, :]
    cur_b = rs_half(0, 1)

    fo_h = [None, None]
    fl_h = [None, None]
    for t in range(N_DEV - 1):
        c = lax.rem(p - t - 1 + N_DEV, N_DEV)
        compute_half(c, 0)
        cur_a[0].wait()
        cur_a[1].wait()
        if t < N_DEV - 2:
            comm_o[t + 1, pl.ds(0, HCH)] = (
                comm_o[t + 1, pl.ds(0, HCH)].astype(jnp.float32)
                + work_o[0:HCH, :]).astype(jnp.bfloat16)
            comm_l[t + 1, pl.ds(0, HCH)] = (
                comm_l[t + 1, pl.ds(0, HCH)] + work_l[0:HCH, :])
            nxt_a = rs_half(t + 1, 0)
        else:
            fo_h[0] = comm_o[t + 1, pl.ds(0, HCH)].astype(
                jnp.float32) + work_o[0:HCH, :]
            fl_h[0] = comm_l[t + 1, pl.ds(0, HCH)] + work_l[0:HCH, :]
        compute_half(c, 1)
        cur_b[0].wait()
        cur_b[1].wait()
        if t < N_DEV - 2:
            comm_o[t + 1, pl.ds(HCH, HCH)] = (
                comm_o[t + 1, pl.ds(HCH, HCH)].astype(jnp.float32)
                + work_o[HCH:RCH, :]).astype(jnp.bfloat16)
            comm_l[t + 1, pl.ds(HCH, HCH)] = (
                comm_l[t + 1, pl.ds(HCH, HCH)] + work_l[HCH:RCH, :])
            nxt_b = rs_half(t + 1, 1)
            cur_a, cur_b = nxt_a, nxt_b
        else:
            fo_h[1] = comm_o[t + 1, pl.ds(HCH, HCH)].astype(
                jnp.float32) + work_o[HCH:RCH, :]
            fl_h[1] = comm_l[t + 1, pl.ds(HCH, HCH)] + work_l[HCH:RCH, :]

    for j in range(2):
        inv = 1.0 / fl_h[j]
        for h in range(HQ):
            ctx_ref[j * HCH:(j + 1) * HCH, h * DH:(h + 1) * DH] = (
                fo_h[j][:, h * DH:(h + 1) * DH] * inv[:, h:h + 1]
            ).astype(jnp.bfloat16)
    wo = wo_ref[:].astype(jnp.bfloat16)
    my_out = lax.dot_general(ctx_ref[:], wo, (((1,), (0,)), ((), ())),
                             preferred_element_type=jnp.float32)
    my_chunk = lax.rem(p + 1, N_DEV)
    out_ref[0, pl.ds(my_chunk * RCH, RCH), :] = my_out
    comm_g[0] = my_out.astype(jnp.bfloat16)

    ag_r = pltpu.make_async_remote_copy(
        src_ref=comm_g.at[0], dst_ref=comm_g.at[1],
        send_sem=send_g.at[0], recv_sem=recv_g.at[1],
        device_id=(right,), device_id_type=pl.DeviceIdType.MESH)
    ag_l = pltpu.make_async_remote_copy(
        src_ref=comm_g.at[0], dst_ref=comm_g.at[2],
        send_sem=send_g.at[1], recv_sem=recv_g.at[2],
        device_id=(left,), device_id_type=pl.DeviceIdType.MESH)
    ag_r.start()
    ag_l.start()
    ag_l.wait()
    ag_fl = pltpu.make_async_remote_copy(
        src_ref=comm_g.at[2, pl.ds(0, HCH)],
        dst_ref=comm_g.at[3, pl.ds(0, HCH)],
        send_sem=send_g.at[2], recv_sem=recv_g.at[3],
        device_id=(left,), device_id_type=pl.DeviceIdType.MESH)
    ag_fl.start()
    out_ref[0, pl.ds(lax.rem(p + 2, N_DEV) * RCH, RCH), :] = (
        comm_g[2].astype(jnp.float32))
    ag_r.wait()
    ag_fr = pltpu.make_async_remote_copy(
        src_ref=comm_g.at[1, pl.ds(HCH, HCH)],
        dst_ref=comm_g.at[3, pl.ds(HCH, HCH)],
        send_sem=send_g.at[3], recv_sem=recv_g.at[4],
        device_id=(right,), device_id_type=pl.DeviceIdType.MESH)
    ag_fr.start()
    out_ref[0, pl.ds(p * RCH, RCH), :] = comm_g[1].astype(jnp.float32)
    ag_fl.wait()
    ag_fr.wait()
    out_ref[0, pl.ds(lax.rem(p + 3, N_DEV) * RCH, RCH), :] = (
        comm_g[3].astype(jnp.float32))


def kernel(x, Wq, K_ext, V_ext, Wo):
    Q3 = pl.pallas_call(
        _qproj_body,
        out_shape=jax.ShapeDtypeStruct((HQ, SQ, DH), jnp.bfloat16),
        in_specs=[pl.BlockSpec(memory_space=pltpu.VMEM),
                  pl.BlockSpec(memory_space=pltpu.VMEM)],
        out_specs=pl.BlockSpec(memory_space=pltpu.VMEM),
    )(x, Wq)

    K_t, V_t = pl.pallas_call(
        _kvprep_body,
        out_shape=[
            jax.ShapeDtypeStruct((HQ, SKV_LOC, DH), jnp.bfloat16),
            jax.ShapeDtypeStruct((HQ, SKV_LOC, DH), jnp.bfloat16),
        ],
        in_specs=[pl.BlockSpec(memory_space=pltpu.VMEM),
                  pl.BlockSpec(memory_space=pltpu.VMEM)],
        out_specs=[pl.BlockSpec(memory_space=pltpu.VMEM),
                   pl.BlockSpec(memory_space=pltpu.VMEM)],
    )(K_ext, V_ext)

    out = pl.pallas_call(
        _fused_body,
        out_shape=jax.ShapeDtypeStruct((1, SQ, DM), jnp.float32),
        in_specs=[pl.BlockSpec(memory_space=pltpu.VMEM),
                  pl.BlockSpec(memory_space=pltpu.VMEM),
                  pl.BlockSpec(memory_space=pltpu.VMEM),
                  pl.BlockSpec(memory_space=pltpu.VMEM)],
        out_specs=pl.BlockSpec(memory_space=pltpu.VMEM),
        scratch_shapes=[
            pltpu.VMEM((HCH, SKV_LOC), jnp.bfloat16),
            pltpu.VMEM((N_DEV, RCH, DM), jnp.bfloat16),
            pltpu.VMEM((N_DEV, RCH, HQ), jnp.float32),
            pltpu.VMEM((N_DEV, RCH, DM), jnp.bfloat16),
            pltpu.VMEM((RCH, DM), jnp.float32),
            pltpu.VMEM((RCH, HQ), jnp.float32),
            pltpu.VMEM((RCH, DM), jnp.bfloat16),
            pltpu.SemaphoreType.DMA((8,)),
            pltpu.SemaphoreType.DMA((8,)),
            pltpu.SemaphoreType.DMA((8,)),
            pltpu.SemaphoreType.DMA((8,)),
            pltpu.SemaphoreType.DMA((8,)),
            pltpu.SemaphoreType.DMA((8,)),
        ],
        compiler_params=pltpu.CompilerParams(collective_id=0),
    )(Q3, K_t, V_t, Wo)
    return out
